# Initial kernel scaffold; baseline (speedup 1.0000x reference)
import jax
import jax.numpy as jnp
from jax.experimental import pallas as pl


def kernel(video_1_fea, video_2_fea, video_1_fused, video_2_fused, Wl1, bl1, Wr1, br1, att1, bias1, Wl2, bl2, Wr2, br2, att2, bias2, Wo, bo, ln_g, ln_b, res_w):
    raise NotImplementedError("write your pallas kernel here")


# speedup vs baseline: 54.6520x; 54.6520x over previous
import jax
import jax.numpy as jnp
from jax.experimental import pallas as pl
from jax.experimental.pallas import tpu as pltpu

N = 256
NOUT = 128
HEADS = 4
HC1 = 256
HC2 = 128
DBLK = 8


def _attn_layer(xl_ref, xr_ref, att_ref, out_ref, nd, hc):
    xl_h = [xl_ref[:, h * hc:(h + 1) * hc] for h in range(HEADS)]
    att_h = [att_ref[h:h + 1, :] for h in range(HEADS)]
    al_h = [jax.lax.dot_general(att_h[h], xl_h[h], (((1,), (1,)), ((), ())),
                                preferred_element_type=jnp.float32)
            for h in range(HEADS)]

    def body(b, carry):
        d0 = b * DBLK
        for h in range(HEADS):
            xrb = xr_ref[pl.ds(d0, DBLK), h * hc:(h + 1) * hc]
            ar = jax.lax.dot_general(xrb, att_h[h], (((1,), (1,)), ((), ())),
                                     preferred_element_type=jnp.float32)
            t = xrb[:, None, :] + xl_h[h][None, :, :]
            sc = jnp.sum(jnp.abs(t) * (0.4 * att_h[h])[None, :, :], axis=-1)
            sc = sc + 0.6 * (ar + al_h[h])
            rows = d0 + jax.lax.broadcasted_iota(jnp.int32, (DBLK, N), 0)
            cols = jax.lax.broadcasted_iota(jnp.int32, (DBLK, N), 1)
            sc = jnp.where(rows == cols, -1e30, sc)
            m = jnp.max(sc, axis=1, keepdims=True)
            e = jnp.exp(sc - m)
            den = jnp.sum(e, axis=1, keepdims=True)
            a = e / (den + 1e-16)
            ob = jnp.dot(a, xl_h[h], preferred_element_type=jnp.float32)
            out_ref[pl.ds(d0, DBLK), h * hc:(h + 1) * hc] = ob
        return carry

    jax.lax.fori_loop(0, nd // DBLK, body, 0)


def _body(x_ref, wlt1_ref, bl1_ref, wrt1_ref, br1_ref, att1_ref, bias1_ref,
          wlt2_ref, bl2_ref, wrt2_ref, br2_ref, att2_ref, bias2_ref,
          wot_ref, bo_ref, lng_ref, lnb_ref, resw_ref,
          out_ref,
          xl1_ref, xr1_ref, agg1_ref, xl2_ref, xr2_ref, agg2_ref):
    x = x_ref[...]
    xl1_ref[...] = jnp.dot(x, wlt1_ref[...], preferred_element_type=jnp.float32) + bl1_ref[...]
    xr1_ref[...] = jnp.dot(x, wrt1_ref[...], preferred_element_type=jnp.float32) + br1_ref[...]
    _attn_layer(xl1_ref, xr1_ref, att1_ref, agg1_ref, N, HC1)

    o1 = agg1_ref[...] + bias1_ref[...]
    y = jnp.where(o1 > 0, o1, jnp.exp(jnp.minimum(o1, 0.0)) - 1.0)
    xl2_ref[...] = jnp.dot(y, wlt2_ref[...], preferred_element_type=jnp.float32) + bl2_ref[...]
    xr2_ref[...] = jnp.dot(y[:NOUT], wrt2_ref[...], preferred_element_type=jnp.float32) + br2_ref[...]
    _attn_layer(xl2_ref, xr2_ref, att2_ref, agg2_ref, NOUT, HC2)

    o2 = agg2_ref[...] + bias2_ref[...]
    h = jnp.dot(o2, wot_ref[...], preferred_element_type=jnp.float32) + bo_ref[...]
    mu = jnp.mean(h, axis=-1, keepdims=True)
    var = jnp.mean((h - mu) ** 2, axis=-1, keepdims=True)
    h = (h - mu) * jax.lax.rsqrt(var + 1e-5) * lng_ref[...] + lnb_ref[...]
    h = jnp.maximum(h, 0.0)
    out_ref[...] = h + resw_ref[0, 0] * x[:NOUT]


def kernel(video_1_fea, video_2_fea, video_1_fused, video_2_fused,
           Wl1, bl1, Wr1, br1, att1, bias1,
           Wl2, bl2, Wr2, br2, att2, bias2,
           Wo, bo, ln_g, ln_b, res_w):
    B1, T, C = video_1_fea.shape
    B2 = video_2_fea.shape[0]
    x = jnp.concatenate([
        video_1_fea.reshape(B1 * T, C),
        video_2_fea.reshape(B2 * T, C),
        video_1_fused.reshape(B1 * T, C),
        video_2_fused.reshape(B2 * T, C),
    ], axis=0)

    f32 = jnp.float32
    h = pl.pallas_call(
        _body,
        out_shape=jax.ShapeDtypeStruct((NOUT, C), f32),
        scratch_shapes=[
            pltpu.VMEM((N, HEADS * HC1), f32),
            pltpu.VMEM((N, HEADS * HC1), f32),
            pltpu.VMEM((N, HEADS * HC1), f32),
            pltpu.VMEM((N, HEADS * HC2), f32),
            pltpu.VMEM((NOUT, HEADS * HC2), f32),
            pltpu.VMEM((NOUT, HEADS * HC2), f32),
        ],
    )(x,
      Wl1.T, bl1.reshape(1, -1), Wr1.T, br1.reshape(1, -1), att1, bias1.reshape(1, -1),
      Wl2.T, bl2.reshape(1, -1), Wr2.T, br2.reshape(1, -1), att2, bias2.reshape(1, -1),
      Wo.T, bo.reshape(1, -1), ln_g.reshape(1, -1), ln_b.reshape(1, -1),
      res_w.reshape(1, 1))

    p1 = h[:B1 * T].reshape(B1, T, C)
    p2 = h[B1 * T:].reshape(B2, T, C)
    return jnp.concatenate([p1, p2], axis=0)


# speedup vs baseline: 59.6236x; 1.0910x over previous
import jax
import jax.numpy as jnp
from jax.experimental import pallas as pl
from jax.experimental.pallas import tpu as pltpu

N = 256
NOUT = 128
HEADS = 4
HC1 = 256
HC2 = 128
DBLK = 16


def _attn_layer(xl_ref, xr_ref, att_ref, out_ref, nd, hc):
    xl_h = [xl_ref[:, h * hc:(h + 1) * hc] for h in range(HEADS)]
    att_h = [att_ref[h:h + 1, :] for h in range(HEADS)]
    al_h = [jax.lax.dot_general(att_h[h], xl_h[h], (((1,), (1,)), ((), ())),
                                preferred_element_type=jnp.float32)
            for h in range(HEADS)]

    def body(b, carry):
        d0 = b * DBLK
        for h in range(HEADS):
            xrb = xr_ref[pl.ds(d0, DBLK), h * hc:(h + 1) * hc]
            ar = jax.lax.dot_general(xrb, att_h[h], (((1,), (1,)), ((), ())),
                                     preferred_element_type=jnp.float32)
            t = xrb[:, None, :] + xl_h[h][None, :, :]
            sc = jnp.sum(jnp.abs(t) * (0.4 * att_h[h])[None, :, :], axis=-1)
            sc = sc + 0.6 * (ar + al_h[h])
            rows = d0 + jax.lax.broadcasted_iota(jnp.int32, (DBLK, N), 0)
            cols = jax.lax.broadcasted_iota(jnp.int32, (DBLK, N), 1)
            sc = jnp.where(rows == cols, -1e30, sc)
            m = jnp.max(sc, axis=1, keepdims=True)
            e = jnp.exp(sc - m)
            den = jnp.sum(e, axis=1, keepdims=True)
            a = e / (den + 1e-16)
            ob = jnp.dot(a, xl_h[h], preferred_element_type=jnp.float32)
            out_ref[pl.ds(d0, DBLK), h * hc:(h + 1) * hc] = ob
        return carry

    jax.lax.fori_loop(0, nd // DBLK, body, 0)


def _body(x_ref, wlt1_ref, bl1_ref, wrt1_ref, br1_ref, att1_ref, bias1_ref,
          wlt2_ref, bl2_ref, wrt2_ref, br2_ref, att2_ref, bias2_ref,
          wot_ref, bo_ref, lng_ref, lnb_ref, resw_ref,
          out_ref,
          xl1_ref, xr1_ref, agg1_ref, xl2_ref, xr2_ref, agg2_ref):
    x = x_ref[...]
    xl1_ref[...] = jnp.dot(x, wlt1_ref[...], preferred_element_type=jnp.float32) + bl1_ref[...]
    xr1_ref[...] = jnp.dot(x, wrt1_ref[...], preferred_element_type=jnp.float32) + br1_ref[...]
    _attn_layer(xl1_ref, xr1_ref, att1_ref, agg1_ref, N, HC1)

    o1 = agg1_ref[...] + bias1_ref[...]
    y = jnp.where(o1 > 0, o1, jnp.exp(jnp.minimum(o1, 0.0)) - 1.0)
    xl2_ref[...] = jnp.dot(y, wlt2_ref[...], preferred_element_type=jnp.float32) + bl2_ref[...]
    xr2_ref[...] = jnp.dot(y[:NOUT], wrt2_ref[...], preferred_element_type=jnp.float32) + br2_ref[...]
    _attn_layer(xl2_ref, xr2_ref, att2_ref, agg2_ref, NOUT, HC2)

    o2 = agg2_ref[...] + bias2_ref[...]
    h = jnp.dot(o2, wot_ref[...], preferred_element_type=jnp.float32) + bo_ref[...]
    mu = jnp.mean(h, axis=-1, keepdims=True)
    var = jnp.mean((h - mu) ** 2, axis=-1, keepdims=True)
    h = (h - mu) * jax.lax.rsqrt(var + 1e-5) * lng_ref[...] + lnb_ref[...]
    h = jnp.maximum(h, 0.0)
    out_ref[...] = h + resw_ref[0, 0] * x[:NOUT]


def kernel(video_1_fea, video_2_fea, video_1_fused, video_2_fused,
           Wl1, bl1, Wr1, br1, att1, bias1,
           Wl2, bl2, Wr2, br2, att2, bias2,
           Wo, bo, ln_g, ln_b, res_w):
    B1, T, C = video_1_fea.shape
    B2 = video_2_fea.shape[0]
    x = jnp.concatenate([
        video_1_fea.reshape(B1 * T, C),
        video_2_fea.reshape(B2 * T, C),
        video_1_fused.reshape(B1 * T, C),
        video_2_fused.reshape(B2 * T, C),
    ], axis=0)

    f32 = jnp.float32
    h = pl.pallas_call(
        _body,
        out_shape=jax.ShapeDtypeStruct((NOUT, C), f32),
        scratch_shapes=[
            pltpu.VMEM((N, HEADS * HC1), f32),
            pltpu.VMEM((N, HEADS * HC1), f32),
            pltpu.VMEM((N, HEADS * HC1), f32),
            pltpu.VMEM((N, HEADS * HC2), f32),
            pltpu.VMEM((NOUT, HEADS * HC2), f32),
            pltpu.VMEM((NOUT, HEADS * HC2), f32),
        ],
    )(x,
      Wl1.T, bl1.reshape(1, -1), Wr1.T, br1.reshape(1, -1), att1, bias1.reshape(1, -1),
      Wl2.T, bl2.reshape(1, -1), Wr2.T, br2.reshape(1, -1), att2, bias2.reshape(1, -1),
      Wo.T, bo.reshape(1, -1), ln_g.reshape(1, -1), ln_b.reshape(1, -1),
      res_w.reshape(1, 1))

    p1 = h[:B1 * T].reshape(B1, T, C)
    p2 = h[B1 * T:].reshape(B2, T, C)
    return jnp.concatenate([p1, p2], axis=0)


# speedup vs baseline: 62.9877x; 1.0564x over previous
import jax
import jax.numpy as jnp
from jax.experimental import pallas as pl
from jax.experimental.pallas import tpu as pltpu

N = 256
NOUT = 128
HEADS = 4
HC1 = 256
HC2 = 128
DBLK = 32


def _attn_layer(xl_ref, xr_ref, att_ref, out_ref, nd, hc):
    xl_h = [xl_ref[:, h * hc:(h + 1) * hc] for h in range(HEADS)]
    att_h = [att_ref[h:h + 1, :] for h in range(HEADS)]
    al_h = [jax.lax.dot_general(att_h[h], xl_h[h], (((1,), (1,)), ((), ())),
                                preferred_element_type=jnp.float32)
            for h in range(HEADS)]

    def body(b, carry):
        d0 = b * DBLK
        for h in range(HEADS):
            xrb = xr_ref[pl.ds(d0, DBLK), h * hc:(h + 1) * hc]
            ar = jax.lax.dot_general(xrb, att_h[h], (((1,), (1,)), ((), ())),
                                     preferred_element_type=jnp.float32)
            t = xrb[:, None, :] + xl_h[h][None, :, :]
            sc = jnp.sum(jnp.abs(t) * (0.4 * att_h[h])[None, :, :], axis=-1)
            sc = sc + 0.6 * (ar + al_h[h])
            rows = d0 + jax.lax.broadcasted_iota(jnp.int32, (DBLK, N), 0)
            cols = jax.lax.broadcasted_iota(jnp.int32, (DBLK, N), 1)
            sc = jnp.where(rows == cols, -1e30, sc)
            m = jnp.max(sc, axis=1, keepdims=True)
            e = jnp.exp(sc - m)
            den = jnp.sum(e, axis=1, keepdims=True)
            a = e / (den + 1e-16)
            ob = jnp.dot(a, xl_h[h], preferred_element_type=jnp.float32)
            out_ref[pl.ds(d0, DBLK), h * hc:(h + 1) * hc] = ob
        return carry

    jax.lax.fori_loop(0, nd // DBLK, body, 0)


def _body(x_ref, wlt1_ref, bl1_ref, wrt1_ref, br1_ref, att1_ref, bias1_ref,
          wlt2_ref, bl2_ref, wrt2_ref, br2_ref, att2_ref, bias2_ref,
          wot_ref, bo_ref, lng_ref, lnb_ref, resw_ref,
          out_ref,
          xl1_ref, xr1_ref, agg1_ref, xl2_ref, xr2_ref, agg2_ref):
    x = x_ref[...]
    xl1_ref[...] = jnp.dot(x, wlt1_ref[...], preferred_element_type=jnp.float32) + bl1_ref[...]
    xr1_ref[...] = jnp.dot(x, wrt1_ref[...], preferred_element_type=jnp.float32) + br1_ref[...]
    _attn_layer(xl1_ref, xr1_ref, att1_ref, agg1_ref, N, HC1)

    o1 = agg1_ref[...] + bias1_ref[...]
    y = jnp.where(o1 > 0, o1, jnp.exp(jnp.minimum(o1, 0.0)) - 1.0)
    xl2_ref[...] = jnp.dot(y, wlt2_ref[...], preferred_element_type=jnp.float32) + bl2_ref[...]
    xr2_ref[...] = jnp.dot(y[:NOUT], wrt2_ref[...], preferred_element_type=jnp.float32) + br2_ref[...]
    _attn_layer(xl2_ref, xr2_ref, att2_ref, agg2_ref, NOUT, HC2)

    o2 = agg2_ref[...] + bias2_ref[...]
    h = jnp.dot(o2, wot_ref[...], preferred_element_type=jnp.float32) + bo_ref[...]
    mu = jnp.mean(h, axis=-1, keepdims=True)
    var = jnp.mean((h - mu) ** 2, axis=-1, keepdims=True)
    h = (h - mu) * jax.lax.rsqrt(var + 1e-5) * lng_ref[...] + lnb_ref[...]
    h = jnp.maximum(h, 0.0)
    out_ref[...] = h + resw_ref[0, 0] * x[:NOUT]


def kernel(video_1_fea, video_2_fea, video_1_fused, video_2_fused,
           Wl1, bl1, Wr1, br1, att1, bias1,
           Wl2, bl2, Wr2, br2, att2, bias2,
           Wo, bo, ln_g, ln_b, res_w):
    B1, T, C = video_1_fea.shape
    B2 = video_2_fea.shape[0]
    x = jnp.concatenate([
        video_1_fea.reshape(B1 * T, C),
        video_2_fea.reshape(B2 * T, C),
        video_1_fused.reshape(B1 * T, C),
        video_2_fused.reshape(B2 * T, C),
    ], axis=0)

    f32 = jnp.float32
    h = pl.pallas_call(
        _body,
        out_shape=jax.ShapeDtypeStruct((NOUT, C), f32),
        scratch_shapes=[
            pltpu.VMEM((N, HEADS * HC1), f32),
            pltpu.VMEM((N, HEADS * HC1), f32),
            pltpu.VMEM((N, HEADS * HC1), f32),
            pltpu.VMEM((N, HEADS * HC2), f32),
            pltpu.VMEM((NOUT, HEADS * HC2), f32),
            pltpu.VMEM((NOUT, HEADS * HC2), f32),
        ],
    )(x,
      Wl1.T, bl1.reshape(1, -1), Wr1.T, br1.reshape(1, -1), att1, bias1.reshape(1, -1),
      Wl2.T, bl2.reshape(1, -1), Wr2.T, br2.reshape(1, -1), att2, bias2.reshape(1, -1),
      Wo.T, bo.reshape(1, -1), ln_g.reshape(1, -1), ln_b.reshape(1, -1),
      res_w.reshape(1, 1))

    p1 = h[:B1 * T].reshape(B1, T, C)
    p2 = h[B1 * T:].reshape(B2, T, C)
    return jnp.concatenate([p1, p2], axis=0)


# speedup vs baseline: 92.1013x; 1.4622x over previous
import jax
import jax.numpy as jnp
from jax.experimental import pallas as pl
from jax.experimental.pallas import tpu as pltpu

N = 256
NOUT = 128
HEADS = 4
HC1 = 256
HC2 = 128
DBLK = 32


def _attn_layer(xl_ref, xr_ref, att_ref, out_ref, nd, hc):
    xl_h = [xl_ref[:, h * hc:(h + 1) * hc] for h in range(HEADS)]
    att4_h = [0.4 * att_ref[h:h + 1, :] for h in range(HEADS)]
    xlT_h = [jnp.transpose(xl_h[h] * att4_h[h]) for h in range(HEADS)]
    sgn_h = [jnp.broadcast_to(jnp.sign(jnp.transpose(att4_h[h])), (hc, xl_ref.shape[0]))
             for h in range(HEADS)]
    al_h = [jnp.sum(xlT_h[h], axis=0, keepdims=True) for h in range(HEADS)]

    def body(b, carry):
        d0 = b * DBLK
        for h in range(HEADS):
            xrb = xr_ref[pl.ds(d0, DBLK), h * hc:(h + 1) * hc] * att4_h[h]
            ar = jnp.sum(xrb, axis=1, keepdims=True)
            t = xrb[:, :, None] + xlT_h[h][None, :, :]
            sc = jnp.sum(jnp.abs(t) * sgn_h[h][None, :, :], axis=1)
            sc = sc + 1.5 * (ar + al_h[h])
            rows = d0 + jax.lax.broadcasted_iota(jnp.int32, (DBLK, N), 0)
            cols = jax.lax.broadcasted_iota(jnp.int32, (DBLK, N), 1)
            sc = jnp.where(rows == cols, -1e30, sc)
            m = jnp.max(sc, axis=1, keepdims=True)
            e = jnp.exp(sc - m)
            den = jnp.sum(e, axis=1, keepdims=True)
            a = e / (den + 1e-16)
            ob = jnp.dot(a, xl_h[h], preferred_element_type=jnp.float32)
            out_ref[pl.ds(d0, DBLK), h * hc:(h + 1) * hc] = ob
        return carry

    jax.lax.fori_loop(0, nd // DBLK, body, 0)


def _body(x_ref, wlt1_ref, bl1_ref, wrt1_ref, br1_ref, att1_ref, bias1_ref,
          wlt2_ref, bl2_ref, wrt2_ref, br2_ref, att2_ref, bias2_ref,
          wot_ref, bo_ref, lng_ref, lnb_ref, resw_ref,
          out_ref,
          xl1_ref, xr1_ref, agg1_ref, xl2_ref, xr2_ref, agg2_ref):
    x = x_ref[...]
    xl1_ref[...] = jnp.dot(x, wlt1_ref[...], preferred_element_type=jnp.float32) + bl1_ref[...]
    xr1_ref[...] = jnp.dot(x, wrt1_ref[...], preferred_element_type=jnp.float32) + br1_ref[...]
    _attn_layer(xl1_ref, xr1_ref, att1_ref, agg1_ref, N, HC1)

    o1 = agg1_ref[...] + bias1_ref[...]
    y = jnp.where(o1 > 0, o1, jnp.exp(jnp.minimum(o1, 0.0)) - 1.0)
    xl2_ref[...] = jnp.dot(y, wlt2_ref[...], preferred_element_type=jnp.float32) + bl2_ref[...]
    xr2_ref[...] = jnp.dot(y[:NOUT], wrt2_ref[...], preferred_element_type=jnp.float32) + br2_ref[...]
    _attn_layer(xl2_ref, xr2_ref, att2_ref, agg2_ref, NOUT, HC2)

    o2 = agg2_ref[...] + bias2_ref[...]
    h = jnp.dot(o2, wot_ref[...], preferred_element_type=jnp.float32) + bo_ref[...]
    mu = jnp.mean(h, axis=-1, keepdims=True)
    var = jnp.mean((h - mu) ** 2, axis=-1, keepdims=True)
    h = (h - mu) * jax.lax.rsqrt(var + 1e-5) * lng_ref[...] + lnb_ref[...]
    h = jnp.maximum(h, 0.0)
    out_ref[...] = h + resw_ref[0, 0] * x[:NOUT]


def kernel(video_1_fea, video_2_fea, video_1_fused, video_2_fused,
           Wl1, bl1, Wr1, br1, att1, bias1,
           Wl2, bl2, Wr2, br2, att2, bias2,
           Wo, bo, ln_g, ln_b, res_w):
    B1, T, C = video_1_fea.shape
    B2 = video_2_fea.shape[0]
    x = jnp.concatenate([
        video_1_fea.reshape(B1 * T, C),
        video_2_fea.reshape(B2 * T, C),
        video_1_fused.reshape(B1 * T, C),
        video_2_fused.reshape(B2 * T, C),
    ], axis=0)

    f32 = jnp.float32
    h = pl.pallas_call(
        _body,
        out_shape=jax.ShapeDtypeStruct((NOUT, C), f32),
        scratch_shapes=[
            pltpu.VMEM((N, HEADS * HC1), f32),
            pltpu.VMEM((N, HEADS * HC1), f32),
            pltpu.VMEM((N, HEADS * HC1), f32),
            pltpu.VMEM((N, HEADS * HC2), f32),
            pltpu.VMEM((NOUT, HEADS * HC2), f32),
            pltpu.VMEM((NOUT, HEADS * HC2), f32),
        ],
    )(x,
      Wl1.T, bl1.reshape(1, -1), Wr1.T, br1.reshape(1, -1), att1, bias1.reshape(1, -1),
      Wl2.T, bl2.reshape(1, -1), Wr2.T, br2.reshape(1, -1), att2, bias2.reshape(1, -1),
      Wo.T, bo.reshape(1, -1), ln_g.reshape(1, -1), ln_b.reshape(1, -1),
      res_w.reshape(1, 1))

    p1 = h[:B1 * T].reshape(B1, T, C)
    p2 = h[B1 * T:].reshape(B2, T, C)
    return jnp.concatenate([p1, p2], axis=0)


# speedup vs baseline: 113.4543x; 1.2318x over previous
import jax
import jax.numpy as jnp
from jax.experimental import pallas as pl
from jax.experimental.pallas import tpu as pltpu

N = 256
NOUT = 128
HEADS = 4
HC1 = 256
HC2 = 128
DBLK = 32
NCORE = 2


def _attn(xl, xr, att_ref, nd, hc, d_base):
    outs = []
    for h in range(HEADS):
        xl_h = xl[:, h * hc:(h + 1) * hc]
        att4 = 0.4 * att_ref[h:h + 1, :]
        xlT = jnp.transpose(xl_h * att4)
        sgn = jnp.broadcast_to(jnp.sign(jnp.transpose(att4)), (hc, N))
        al = jnp.sum(xlT, axis=0, keepdims=True)
        xr_h = xr[:, h * hc:(h + 1) * hc] * att4
        ar = jnp.sum(xr_h, axis=1, keepdims=True)

        blocks = []
        for b in range(nd // DBLK):
            d0 = b * DBLK
            xrb = xr_h[d0:d0 + DBLK]
            t = xrb[:, :, None] + xlT[None, :, :]
            sc = jnp.sum(jnp.abs(t) * sgn[None, :, :], axis=1)
            sc = sc + 1.5 * (ar[d0:d0 + DBLK] + al)
            rows = d_base + d0 + jax.lax.broadcasted_iota(jnp.int32, (DBLK, N), 0)
            cols = jax.lax.broadcasted_iota(jnp.int32, (DBLK, N), 1)
            sc = jnp.where(rows == cols, -1e30, sc)
            m = jnp.max(sc, axis=1, keepdims=True)
            e = jnp.exp(sc - m)
            den = jnp.sum(e, axis=1, keepdims=True)
            a = e / (den + 1e-16)
            blocks.append(jnp.dot(a, xl_h, preferred_element_type=jnp.float32))
        outs.append(jnp.concatenate(blocks, axis=0) if len(blocks) > 1 else blocks[0])
    return jnp.concatenate(outs, axis=1)


def _body1(x_ref, wlt1_ref, bl1_ref, wrt1_ref, br1_ref, att1_ref, agg1_ref):
    i = pl.program_id(0)
    nd = N // NCORE
    x = x_ref[...]
    xl1 = jnp.dot(x, wlt1_ref[...], preferred_element_type=jnp.float32) + bl1_ref[...]
    xh = x_ref[pl.ds(i * nd, nd), :]
    xr1 = jnp.dot(xh, wrt1_ref[...], preferred_element_type=jnp.float32) + br1_ref[...]
    agg1_ref[...] = _attn(xl1, xr1, att1_ref, nd, HC1, i * nd)


def _body2(x_ref, agg1_ref, bias1_ref, wlt2_ref, bl2_ref, wrt2_ref, br2_ref,
           att2_ref, bias2_ref, wot_ref, bo_ref, lng_ref, lnb_ref, resw_ref,
           out_ref):
    i = pl.program_id(0)
    nd = NOUT // NCORE
    o1 = agg1_ref[...] + bias1_ref[...]
    y = jnp.where(o1 > 0, o1, jnp.exp(jnp.minimum(o1, 0.0)) - 1.0)
    xl2 = jnp.dot(y, wlt2_ref[...], preferred_element_type=jnp.float32) + bl2_ref[...]
    o1h = agg1_ref[pl.ds(i * nd, nd), :] + bias1_ref[...]
    yh = jnp.where(o1h > 0, o1h, jnp.exp(jnp.minimum(o1h, 0.0)) - 1.0)
    xr2 = jnp.dot(yh, wrt2_ref[...], preferred_element_type=jnp.float32) + br2_ref[...]
    o2 = _attn(xl2, xr2, att2_ref, nd, HC2, i * nd) + bias2_ref[...]
    h = jnp.dot(o2, wot_ref[...], preferred_element_type=jnp.float32) + bo_ref[...]
    mu = jnp.mean(h, axis=-1, keepdims=True)
    var = jnp.mean((h - mu) ** 2, axis=-1, keepdims=True)
    h = (h - mu) * jax.lax.rsqrt(var + 1e-5) * lng_ref[...] + lnb_ref[...]
    h = jnp.maximum(h, 0.0)
    out_ref[...] = h + resw_ref[0, 0] * x_ref[pl.ds(i * nd, nd), :]


def kernel(video_1_fea, video_2_fea, video_1_fused, video_2_fused,
           Wl1, bl1, Wr1, br1, att1, bias1,
           Wl2, bl2, Wr2, br2, att2, bias2,
           Wo, bo, ln_g, ln_b, res_w):
    B1, T, C = video_1_fea.shape
    B2 = video_2_fea.shape[0]
    x = jnp.concatenate([
        video_1_fea.reshape(B1 * T, C),
        video_2_fea.reshape(B2 * T, C),
        video_1_fused.reshape(B1 * T, C),
        video_2_fused.reshape(B2 * T, C),
    ], axis=0)

    f32 = jnp.float32
    full = lambda s: pl.BlockSpec(s, lambda i: (0, 0))
    par = pltpu.CompilerParams(dimension_semantics=("parallel",))

    agg1 = pl.pallas_call(
        _body1,
        grid=(NCORE,),
        in_specs=[full((N, C)), full((C, HEADS * HC1)), full((1, HEADS * HC1)),
                  full((C, HEADS * HC1)), full((1, HEADS * HC1)), full((HEADS, HC1))],
        out_specs=pl.BlockSpec((N // NCORE, HEADS * HC1), lambda i: (i, 0)),
        out_shape=jax.ShapeDtypeStruct((N, HEADS * HC1), f32),
        compiler_params=par,
    )(x, Wl1.T, bl1.reshape(1, -1), Wr1.T, br1.reshape(1, -1), att1)

    h = pl.pallas_call(
        _body2,
        grid=(NCORE,),
        in_specs=[full((N, C)), full((N, HEADS * HC1)), full((1, HEADS * HC1)),
                  full((HEADS * HC1, HEADS * HC2)), full((1, HEADS * HC2)),
                  full((HEADS * HC1, HEADS * HC2)), full((1, HEADS * HC2)),
                  full((HEADS, HC2)), full((1, HEADS * HC2)),
                  full((HEADS * HC2, C)), full((1, C)), full((1, C)), full((1, C)),
                  full((1, 1))],
        out_specs=pl.BlockSpec((NOUT // NCORE, C), lambda i: (i, 0)),
        out_shape=jax.ShapeDtypeStruct((NOUT, C), f32),
        compiler_params=par,
    )(x, agg1, bias1.reshape(1, -1),
      Wl2.T, bl2.reshape(1, -1), Wr2.T, br2.reshape(1, -1), att2, bias2.reshape(1, -1),
      Wo.T, bo.reshape(1, -1), ln_g.reshape(1, -1), ln_b.reshape(1, -1),
      res_w.reshape(1, 1))

    p1 = h[:B1 * T].reshape(B1, T, C)
    p2 = h[B1 * T:].reshape(B2, T, C)
    return jnp.concatenate([p1, p2], axis=0)


# speedup vs baseline: 132.6119x; 1.1689x over previous
import jax
import jax.numpy as jnp
from jax.experimental import pallas as pl
from jax.experimental.pallas import tpu as pltpu

N = 256
NOUT = 128
HEADS = 4
HC1 = 256
HC2 = 128
DBLK = 32
NCORE = 1


def _attn(xl, xr, att_ref, nd, hc, d_base):
    outs = []
    for h in range(HEADS):
        xl_h = xl[:, h * hc:(h + 1) * hc]
        att4 = 0.4 * att_ref[h:h + 1, :]
        xlT = jnp.transpose(xl_h * att4)
        sgn = jnp.broadcast_to(jnp.sign(jnp.transpose(att4)), (hc, N))
        al = jnp.sum(xlT, axis=0, keepdims=True)
        xr_h = xr[:, h * hc:(h + 1) * hc] * att4
        ar = jnp.sum(xr_h, axis=1, keepdims=True)

        blocks = []
        for b in range(nd // DBLK):
            d0 = b * DBLK
            xrb = xr_h[d0:d0 + DBLK]
            t = xrb[:, :, None] + xlT[None, :, :]
            sc = jnp.sum(jnp.abs(t) * sgn[None, :, :], axis=1)
            sc = sc + 1.5 * (ar[d0:d0 + DBLK] + al)
            rows = d_base + d0 + jax.lax.broadcasted_iota(jnp.int32, (DBLK, N), 0)
            cols = jax.lax.broadcasted_iota(jnp.int32, (DBLK, N), 1)
            sc = jnp.where(rows == cols, -1e30, sc)
            m = jnp.max(sc, axis=1, keepdims=True)
            e = jnp.exp(sc - m)
            den = jnp.sum(e, axis=1, keepdims=True)
            a = e / (den + 1e-16)
            blocks.append(jnp.dot(a, xl_h, preferred_element_type=jnp.float32))
        outs.append(jnp.concatenate(blocks, axis=0) if len(blocks) > 1 else blocks[0])
    return jnp.concatenate(outs, axis=1)


def _body1(x_ref, wlt1_ref, bl1_ref, wrt1_ref, br1_ref, att1_ref, agg1_ref):
    i = pl.program_id(0)
    nd = N // NCORE
    x = x_ref[...]
    xl1 = jnp.dot(x, wlt1_ref[...], preferred_element_type=jnp.float32) + bl1_ref[...]
    xh = x_ref[pl.ds(i * nd, nd), :]
    xr1 = jnp.dot(xh, wrt1_ref[...], preferred_element_type=jnp.float32) + br1_ref[...]
    agg1_ref[...] = _attn(xl1, xr1, att1_ref, nd, HC1, i * nd)


def _body2(x_ref, agg1_ref, bias1_ref, wlt2_ref, bl2_ref, wrt2_ref, br2_ref,
           att2_ref, bias2_ref, wot_ref, bo_ref, lng_ref, lnb_ref, resw_ref,
           out_ref):
    i = pl.program_id(0)
    nd = NOUT // NCORE
    o1 = agg1_ref[...] + bias1_ref[...]
    y = jnp.where(o1 > 0, o1, jnp.exp(jnp.minimum(o1, 0.0)) - 1.0)
    xl2 = jnp.dot(y, wlt2_ref[...], preferred_element_type=jnp.float32) + bl2_ref[...]
    o1h = agg1_ref[pl.ds(i * nd, nd), :] + bias1_ref[...]
    yh = jnp.where(o1h > 0, o1h, jnp.exp(jnp.minimum(o1h, 0.0)) - 1.0)
    xr2 = jnp.dot(yh, wrt2_ref[...], preferred_element_type=jnp.float32) + br2_ref[...]
    o2 = _attn(xl2, xr2, att2_ref, nd, HC2, i * nd) + bias2_ref[...]
    h = jnp.dot(o2, wot_ref[...], preferred_element_type=jnp.float32) + bo_ref[...]
    mu = jnp.mean(h, axis=-1, keepdims=True)
    var = jnp.mean((h - mu) ** 2, axis=-1, keepdims=True)
    h = (h - mu) * jax.lax.rsqrt(var + 1e-5) * lng_ref[...] + lnb_ref[...]
    h = jnp.maximum(h, 0.0)
    out_ref[...] = h + resw_ref[0, 0] * x_ref[pl.ds(i * nd, nd), :]


def kernel(video_1_fea, video_2_fea, video_1_fused, video_2_fused,
           Wl1, bl1, Wr1, br1, att1, bias1,
           Wl2, bl2, Wr2, br2, att2, bias2,
           Wo, bo, ln_g, ln_b, res_w):
    B1, T, C = video_1_fea.shape
    B2 = video_2_fea.shape[0]
    x = jnp.concatenate([
        video_1_fea.reshape(B1 * T, C),
        video_2_fea.reshape(B2 * T, C),
        video_1_fused.reshape(B1 * T, C),
        video_2_fused.reshape(B2 * T, C),
    ], axis=0)

    f32 = jnp.float32
    full = lambda s: pl.BlockSpec(s, lambda i: (0, 0))
    par = pltpu.CompilerParams(dimension_semantics=("parallel",))

    agg1 = pl.pallas_call(
        _body1,
        grid=(NCORE,),
        in_specs=[full((N, C)), full((C, HEADS * HC1)), full((1, HEADS * HC1)),
                  full((C, HEADS * HC1)), full((1, HEADS * HC1)), full((HEADS, HC1))],
        out_specs=pl.BlockSpec((N // NCORE, HEADS * HC1), lambda i: (i, 0)),
        out_shape=jax.ShapeDtypeStruct((N, HEADS * HC1), f32),
        compiler_params=par,
    )(x, Wl1.T, bl1.reshape(1, -1), Wr1.T, br1.reshape(1, -1), att1)

    h = pl.pallas_call(
        _body2,
        grid=(NCORE,),
        in_specs=[full((N, C)), full((N, HEADS * HC1)), full((1, HEADS * HC1)),
                  full((HEADS * HC1, HEADS * HC2)), full((1, HEADS * HC2)),
                  full((HEADS * HC1, HEADS * HC2)), full((1, HEADS * HC2)),
                  full((HEADS, HC2)), full((1, HEADS * HC2)),
                  full((HEADS * HC2, C)), full((1, C)), full((1, C)), full((1, C)),
                  full((1, 1))],
        out_specs=pl.BlockSpec((NOUT // NCORE, C), lambda i: (i, 0)),
        out_shape=jax.ShapeDtypeStruct((NOUT, C), f32),
        compiler_params=par,
    )(x, agg1, bias1.reshape(1, -1),
      Wl2.T, bl2.reshape(1, -1), Wr2.T, br2.reshape(1, -1), att2, bias2.reshape(1, -1),
      Wo.T, bo.reshape(1, -1), ln_g.reshape(1, -1), ln_b.reshape(1, -1),
      res_w.reshape(1, 1))

    p1 = h[:B1 * T].reshape(B1, T, C)
    p2 = h[B1 * T:].reshape(B2, T, C)
    return jnp.concatenate([p1, p2], axis=0)


# speedup vs baseline: 135.6747x; 1.0231x over previous
import jax
import jax.numpy as jnp
from jax.experimental import pallas as pl
from jax.experimental.pallas import tpu as pltpu

N = 256
NOUT = 128
HEADS = 4
HC1 = 256
HC2 = 128
DBLK = 32


def _attn(xl, xr, att_ref, nd, hc):
    outs = []
    for h in range(HEADS):
        xl_h = xl[:, h * hc:(h + 1) * hc]
        att4 = 0.4 * att_ref[h:h + 1, :]
        xlT = jnp.transpose(xl_h * att4)
        sgn = jnp.broadcast_to(jnp.sign(jnp.transpose(att4)), (hc, N))
        al = jnp.sum(xlT, axis=0, keepdims=True)
        xr_h = xr[:, h * hc:(h + 1) * hc] * att4
        ar = jnp.sum(xr_h, axis=1, keepdims=True)

        blocks = []
        for b in range(nd // DBLK):
            d0 = b * DBLK
            xrb = xr_h[d0:d0 + DBLK]
            t = xrb[:, :, None] + xlT[None, :, :]
            sc = jnp.sum(jnp.abs(t) * sgn[None, :, :], axis=1)
            sc = sc + 1.5 * (ar[d0:d0 + DBLK] + al)
            rows = d0 + jax.lax.broadcasted_iota(jnp.int32, (DBLK, N), 0)
            cols = jax.lax.broadcasted_iota(jnp.int32, (DBLK, N), 1)
            sc = jnp.where(rows == cols, -1e30, sc)
            m = jnp.max(sc, axis=1, keepdims=True)
            e = jnp.exp(sc - m)
            den = jnp.sum(e, axis=1, keepdims=True)
            a = e / (den + 1e-16)
            blocks.append(jnp.dot(a, xl_h, preferred_element_type=jnp.float32))
        outs.append(jnp.concatenate(blocks, axis=0) if len(blocks) > 1 else blocks[0])
    return jnp.concatenate(outs, axis=1)


def _body(x_ref, wlt1_ref, bl1_ref, wrt1_ref, br1_ref, att1_ref, bias1_ref,
          wlt2_ref, bl2_ref, wrt2_ref, br2_ref, att2_ref, bias2_ref,
          wot_ref, bo_ref, lng_ref, lnb_ref, resw_ref, out_ref):
    x = x_ref[...]
    xl1 = jnp.dot(x, wlt1_ref[...], preferred_element_type=jnp.float32) + bl1_ref[...]
    xr1 = jnp.dot(x, wrt1_ref[...], preferred_element_type=jnp.float32) + br1_ref[...]
    o1 = _attn(xl1, xr1, att1_ref, N, HC1) + bias1_ref[...]
    y = jnp.where(o1 > 0, o1, jnp.exp(jnp.minimum(o1, 0.0)) - 1.0)

    xl2 = jnp.dot(y, wlt2_ref[...], preferred_element_type=jnp.float32) + bl2_ref[...]
    xr2 = jnp.dot(y[:NOUT], wrt2_ref[...], preferred_element_type=jnp.float32) + br2_ref[...]
    o2 = _attn(xl2, xr2, att2_ref, NOUT, HC2) + bias2_ref[...]

    h = jnp.dot(o2, wot_ref[...], preferred_element_type=jnp.float32) + bo_ref[...]
    mu = jnp.mean(h, axis=-1, keepdims=True)
    var = jnp.mean((h - mu) ** 2, axis=-1, keepdims=True)
    h = (h - mu) * jax.lax.rsqrt(var + 1e-5) * lng_ref[...] + lnb_ref[...]
    h = jnp.maximum(h, 0.0)
    out_ref[...] = h + resw_ref[0, 0] * x[:NOUT]


def kernel(video_1_fea, video_2_fea, video_1_fused, video_2_fused,
           Wl1, bl1, Wr1, br1, att1, bias1,
           Wl2, bl2, Wr2, br2, att2, bias2,
           Wo, bo, ln_g, ln_b, res_w):
    B1, T, C = video_1_fea.shape
    B2 = video_2_fea.shape[0]
    x = jnp.concatenate([
        video_1_fea.reshape(B1 * T, C),
        video_2_fea.reshape(B2 * T, C),
        video_1_fused.reshape(B1 * T, C),
        video_2_fused.reshape(B2 * T, C),
    ], axis=0)

    f32 = jnp.float32
    h = pl.pallas_call(
        _body,
        out_shape=jax.ShapeDtypeStruct((NOUT, C), f32),
    )(x, Wl1.T, bl1.reshape(1, -1), Wr1.T, br1.reshape(1, -1), att1, bias1.reshape(1, -1),
      Wl2.T, bl2.reshape(1, -1), Wr2.T, br2.reshape(1, -1), att2, bias2.reshape(1, -1),
      Wo.T, bo.reshape(1, -1), ln_g.reshape(1, -1), ln_b.reshape(1, -1),
      res_w.reshape(1, 1))

    p1 = h[:B1 * T].reshape(B1, T, C)
    p2 = h[B1 * T:].reshape(B2, T, C)
    return jnp.concatenate([p1, p2], axis=0)
